# baseline (speedup 1.0000x reference)
import functools

import jax
import jax.numpy as jnp
from jax import lax
from jax.experimental import pallas as pl
from jax.experimental.pallas import tpu as pltpu
from jax.experimental.pallas import tpu_sc as plsc

EMBED_DIM = 128
N_NODES = 100000

NUM_WORKERS = 32
B_PAD = 100352
B_PER_W = B_PAD // NUM_WORKERS
CHUNK = 224
NCHUNK = B_PER_W // CHUNK

_mesh = plsc.VectorSubcoreMesh(core_axis_name="c", subcore_axis_name="s")


@functools.partial(
    pl.kernel,
    out_type=jax.ShapeDtypeStruct((B_PAD, EMBED_DIM), jnp.float32),
    mesh=_mesh,
    scratch_types=[
        pltpu.VMEM((B_PER_W,), jnp.int32),
        pltpu.VMEM((CHUNK, EMBED_DIM), jnp.float32),
        pltpu.SemaphoreType.DMA,
    ],
)
def _embed_lookup(table_hbm, idx_hbm, out_hbm, idx_v, rows_v, sem):
    wid = lax.axis_index("s") * 2 + lax.axis_index("c")
    base = wid * B_PER_W
    pltpu.sync_copy(idx_hbm.at[pl.ds(base, B_PER_W)], idx_v)
    for k in range(NCHUNK):
        pltpu.async_copy(
            table_hbm.at[idx_v.at[pl.ds(k * CHUNK, CHUNK)]], rows_v, sem
        ).wait()
        pltpu.sync_copy(rows_v, out_hbm.at[pl.ds(base + k * CHUNK, CHUNK)])


def kernel(z, z_embed_weight):
    z = z.astype(jnp.int32)
    z_pad = jnp.pad(z, (0, B_PAD - N_NODES))
    out = _embed_lookup(z_embed_weight, z_pad)
    return out[:N_NODES]


# speedup vs baseline: 1.0174x; 1.0174x over previous
import functools

import jax
import jax.numpy as jnp
from jax import lax
from jax.experimental import pallas as pl
from jax.experimental.pallas import tpu as pltpu
from jax.experimental.pallas import tpu_sc as plsc

EMBED_DIM = 128
N_NODES = 100000

NUM_WORKERS = 32
B_PAD = 100352
B_PER_W = B_PAD // NUM_WORKERS
CHUNK = 224
NCHUNK = B_PER_W // CHUNK

_mesh = plsc.VectorSubcoreMesh(core_axis_name="c", subcore_axis_name="s")


@functools.partial(
    pl.kernel,
    out_type=jax.ShapeDtypeStruct((B_PAD, EMBED_DIM), jnp.float32),
    mesh=_mesh,
    scratch_types=[
        pltpu.VMEM((B_PER_W,), jnp.int32),
        pltpu.VMEM((2, CHUNK, EMBED_DIM), jnp.float32),
        pltpu.SemaphoreType.DMA,
        pltpu.SemaphoreType.DMA,
        pltpu.SemaphoreType.DMA,
        pltpu.SemaphoreType.DMA,
    ],
)
def _embed_lookup(table_hbm, idx_hbm, out_hbm, idx_v, rows_v,
                  gsem0, gsem1, ssem0, ssem1):
    wid = lax.axis_index("s") * 2 + lax.axis_index("c")
    base = wid * B_PER_W
    gsems = (gsem0, gsem1)
    ssems = (ssem0, ssem1)
    pltpu.sync_copy(idx_hbm.at[pl.ds(base, B_PER_W)], idx_v)

    def start_gather(k):
        return pltpu.async_copy(
            table_hbm.at[idx_v.at[pl.ds(k * CHUNK, CHUNK)]],
            rows_v.at[k % 2],
            gsems[k % 2],
        )

    gathers = [None] * NCHUNK
    scatters = [None] * NCHUNK
    gathers[0] = start_gather(0)
    for k in range(NCHUNK):
        if k + 1 < NCHUNK:
            if k >= 1:
                scatters[k - 1].wait()
            gathers[k + 1] = start_gather(k + 1)
        gathers[k].wait()
        scatters[k] = pltpu.async_copy(
            rows_v.at[k % 2],
            out_hbm.at[pl.ds(base + k * CHUNK, CHUNK)],
            ssems[k % 2],
        )
    scatters[NCHUNK - 2].wait()
    scatters[NCHUNK - 1].wait()


def kernel(z, z_embed_weight):
    z = z.astype(jnp.int32)
    z_pad = jnp.pad(z, (0, B_PAD - N_NODES))
    out = _embed_lookup(z_embed_weight, z_pad)
    return out[:N_NODES]


# speedup vs baseline: 1.1362x; 1.1167x over previous
import functools

import jax
import jax.numpy as jnp
from jax import lax
from jax.experimental import pallas as pl
from jax.experimental.pallas import tpu as pltpu
from jax.experimental.pallas import tpu_sc as plsc

EMBED_DIM = 128
TABLE_ROWS = 37
N_NODES = 100000

NUM_WORKERS = 32
PER_W = 3128
PER_W_PAD = 3136
LAST_BASE = N_NODES - PER_W
CHUNK = 320

_PLAN = [(k * CHUNK, CHUNK) for k in range(PER_W // CHUNK)]
_PLAN.append((PER_W - PER_W % CHUNK, PER_W % CHUNK))

_mesh = plsc.VectorSubcoreMesh(core_axis_name="c", subcore_axis_name="s")


@functools.partial(
    pl.kernel,
    out_type=jax.ShapeDtypeStruct((N_NODES * EMBED_DIM,), jnp.float32),
    mesh=_mesh,
    compiler_params=pltpu.CompilerParams(needs_layout_passes=False),
    scratch_types=[
        pltpu.VMEM((TABLE_ROWS * EMBED_DIM,), jnp.float32),
        pltpu.VMEM((PER_W_PAD,), jnp.int32),
        pltpu.VMEM((CHUNK * EMBED_DIM,), jnp.float32),
        pltpu.VMEM((CHUNK * EMBED_DIM,), jnp.float32),
        pltpu.SemaphoreType.DMA,
        pltpu.SemaphoreType.DMA,
    ],
)
def _embed_lookup(table_hbm, idx_hbm, out_hbm, table_v, idx_v, buf0, buf1,
                  ssem0, ssem1):
    wid = lax.axis_index("s") * 2 + lax.axis_index("c")
    base = jnp.minimum(wid * PER_W, LAST_BASE)
    bufs = (buf0, buf1)
    ssems = (ssem0, ssem1)
    pltpu.sync_copy(table_hbm, table_v)
    pltpu.sync_copy(idx_hbm.at[wid], idx_v)

    scatters = [None] * len(_PLAN)
    for k, (off, n) in enumerate(_PLAN):
        if k >= 2:
            scatters[k - 2].wait()
        b = bufs[k % 2]
        groups = (n + 15) // 16

        @plsc.parallel_loop(0, groups)
        def _grp(g):
            zv = idx_v[pl.ds(off + g * 16, 16)]
            src = zv * EMBED_DIM
            dst = (g * 16 + lax.iota(jnp.int32, 16)) * EMBED_DIM

            @plsc.parallel_loop(0, EMBED_DIM, unroll=8)
            def _col(c):
                vals = plsc.load_gather(table_v, [src + c])
                plsc.store_scatter(b, [dst + c], vals)

        scatters[k] = pltpu.async_copy(
            b.at[pl.ds(0, n * EMBED_DIM)],
            out_hbm.at[pl.ds((base + off) * EMBED_DIM, n * EMBED_DIM)],
            ssems[k % 2],
        )
    scatters[-2].wait()
    scatters[-1].wait()


def kernel(z, z_embed_weight):
    z = z.astype(jnp.int32)
    z_pad = jnp.pad(z, (0, LAST_BASE + PER_W_PAD - N_NODES))
    starts = jnp.minimum(jnp.arange(NUM_WORKERS) * PER_W, LAST_BASE)
    idx2d = z_pad[starts[:, None] + jnp.arange(PER_W_PAD)[None, :]]
    out_flat = _embed_lookup(z_embed_weight.reshape(-1), idx2d)
    return out_flat.reshape(N_NODES, EMBED_DIM)


# speedup vs baseline: 3.1872x; 2.8052x over previous
import functools

import jax
import jax.numpy as jnp
from jax import lax
from jax.experimental import pallas as pl
from jax.experimental.pallas import tpu as pltpu
from jax.experimental.pallas import tpu_sc as plsc

EMBED_DIM = 128
TABLE_ROWS = 37
N_NODES = 100000

NUM_WORKERS = 32
PER_W = 3200
STRIDE_W = 3128
LAST_BASE = N_NODES - PER_W
CHUNK = 320
NCHUNK = PER_W // CHUNK
GROUPS = CHUNK // 16

_mesh = plsc.VectorSubcoreMesh(core_axis_name="c", subcore_axis_name="s")


@functools.partial(
    pl.kernel,
    out_type=jax.ShapeDtypeStruct((N_NODES * EMBED_DIM,), jnp.float32),
    mesh=_mesh,
    compiler_params=pltpu.CompilerParams(needs_layout_passes=False),
    scratch_types=[
        pltpu.VMEM((TABLE_ROWS * EMBED_DIM,), jnp.float32),
        pltpu.VMEM((PER_W,), jnp.int32),
        pltpu.VMEM((CHUNK * EMBED_DIM,), jnp.float32),
        pltpu.VMEM((CHUNK * EMBED_DIM,), jnp.float32),
        pltpu.SemaphoreType.DMA,
        pltpu.SemaphoreType.DMA,
    ],
)
def _embed_lookup(table_hbm, idx_hbm, out_hbm, table_v, idx_v, buf0, buf1,
                  ssem0, ssem1):
    wid = lax.axis_index("s") * 2 + lax.axis_index("c")
    base = jnp.minimum(wid * STRIDE_W, LAST_BASE)
    bufs = (buf0, buf1)
    ssems = (ssem0, ssem1)
    pltpu.sync_copy(table_hbm, table_v)
    pltpu.sync_copy(idx_hbm.at[wid], idx_v)

    def fill_and_scatter(k, p):
        b = bufs[p]
        koff = k * CHUNK

        @plsc.parallel_loop(0, GROUPS)
        def _grp(g):
            zv = idx_v[pl.ds(koff + g * 16, 16)]
            for l in range(16):
                src = zv[l] * EMBED_DIM
                dst = (g * 16 + l) * EMBED_DIM
                for j in range(EMBED_DIM // 16):
                    b[pl.ds(dst + j * 16, 16)] = (
                        table_v[pl.ds(src + j * 16, 16)])

        pltpu.async_copy(
            b,
            out_hbm.at[pl.ds((base + koff) * EMBED_DIM, CHUNK * EMBED_DIM)],
            ssems[p],
        )

    def drain(p):
        pltpu.make_async_copy(
            bufs[p],
            out_hbm.at[pl.ds(base * EMBED_DIM, CHUNK * EMBED_DIM)],
            ssems[p],
        ).wait()

    fill_and_scatter(0, 0)
    fill_and_scatter(1, 1)

    @pl.loop(2, NCHUNK, step=2)
    def _chunks(k):
        drain(0)
        fill_and_scatter(k, 0)
        drain(1)
        fill_and_scatter(k + 1, 1)

    drain(0)
    drain(1)


def kernel(z, z_embed_weight):
    z = z.astype(jnp.int32)
    starts = jnp.minimum(jnp.arange(NUM_WORKERS) * STRIDE_W, LAST_BASE)
    idx2d = z[starts[:, None] + jnp.arange(PER_W)[None, :]]
    out_flat = _embed_lookup(z_embed_weight.reshape(-1), idx2d)
    return out_flat.reshape(N_NODES, EMBED_DIM)
